# baseline (speedup 1.0000x reference)
import functools

import jax
import jax.numpy as jnp
from jax import lax
from jax.experimental import pallas as pl
from jax.experimental.pallas import tpu as pltpu
from jax.experimental.pallas import tpu_sc as plsc

_TABLE = 1000
_B = 16384
_NC = 1
_NS = 16
_L = 16
_NW = _NC * _NS
_BPW = _B // _NW


def _body(t_hbm, betas_hbm, alphas_hbm, bars_hbm, out_hbm,
          idx_v, betas_v, alphas_v, bars_v, out_v, sem):
    wid = lax.axis_index("s") * _NC + lax.axis_index("c")
    base = wid * _BPW

    c0 = pltpu.make_async_copy(betas_hbm, betas_v, sem)
    c1 = pltpu.make_async_copy(alphas_hbm, alphas_v, sem)
    c2 = pltpu.make_async_copy(bars_hbm, bars_v, sem)
    c3 = pltpu.make_async_copy(t_hbm.at[pl.ds(base, _BPW)], idx_v, sem)
    c0.start(); c1.start(); c2.start(); c3.start()
    c0.wait(); c1.wait(); c2.wait(); c3.wait()

    lanes3 = lax.iota(jnp.int32, _L) * 3

    def step(j, carry):
        idx = idx_v[pl.ds(j * _L, _L)]
        beta = plsc.load_gather(betas_v, [idx])
        alpha = plsc.load_gather(alphas_v, [idx])
        bar = plsc.load_gather(bars_v, [idx])
        bar = jnp.minimum(jnp.maximum(bar, 0.0), 1.0)
        p = lanes3 + j * (_L * 3)
        plsc.store_scatter(out_v, [p], beta)
        plsc.store_scatter(out_v, [p + 1], alpha)
        plsc.store_scatter(out_v, [p + 2], bar)
        return carry

    half = _BPW // _L // 2
    hw = _BPW // 2 * 3
    lax.fori_loop(0, half, step, 0, unroll=4)
    w0 = pltpu.make_async_copy(
        out_v.at[pl.ds(0, hw)], out_hbm.at[pl.ds(base * 3, hw)], sem)
    w0.start()
    lax.fori_loop(half, 2 * half, step, 0, unroll=4)
    w1 = pltpu.make_async_copy(
        out_v.at[pl.ds(hw, hw)], out_hbm.at[pl.ds(base * 3 + hw, hw)], sem)
    w1.start()
    w0.wait()
    w1.wait()


_ddpm_lookup = functools.partial(
    pl.kernel,
    out_type=jax.ShapeDtypeStruct((_B * 3,), jnp.float32),
    mesh=plsc.VectorSubcoreMesh(core_axis_name="c", subcore_axis_name="s", num_cores=1),
    compiler_params=pltpu.CompilerParams(needs_layout_passes=False),
    scratch_types=[
        pltpu.VMEM((_BPW,), jnp.int32),
        pltpu.VMEM((_TABLE,), jnp.float32),
        pltpu.VMEM((_TABLE,), jnp.float32),
        pltpu.VMEM((_TABLE,), jnp.float32),
        pltpu.VMEM((_BPW * 3,), jnp.float32),
        pltpu.SemaphoreType.DMA,
    ],
)(_body)


@jax.jit
def kernel(T, all_betas, all_alphas, all_bar_alphas):
    flat = _ddpm_lookup(T, all_betas, all_alphas, all_bar_alphas)
    return flat.reshape(_B, 3)


# speedup vs baseline: 1.0049x; 1.0049x over previous
import functools

import jax
import jax.numpy as jnp
from jax import lax
from jax.experimental import pallas as pl
from jax.experimental.pallas import tpu as pltpu
from jax.experimental.pallas import tpu_sc as plsc

_TABLE = 1000
_B = 16384
_NC = 1
_NS = 16
_L = 16
_NW = _NC * _NS
_BPW = _B // _NW


def _body(t_hbm, betas_hbm, alphas_hbm, bars_hbm, out_hbm,
          idx_v, betas_v, alphas_v, bars_v, out_v, sem):
    wid = lax.axis_index("s") * _NC + lax.axis_index("c")
    base = wid * _BPW

    c0 = pltpu.make_async_copy(betas_hbm, betas_v, sem)
    c1 = pltpu.make_async_copy(alphas_hbm, alphas_v, sem)
    c2 = pltpu.make_async_copy(bars_hbm, bars_v, sem)
    c3 = pltpu.make_async_copy(t_hbm.at[pl.ds(base, _BPW)], idx_v, sem)
    c0.start(); c1.start(); c2.start(); c3.start()
    c0.wait(); c1.wait(); c2.wait(); c3.wait()

    lanes3 = lax.iota(jnp.int32, _L) * 3

    def step(j, carry):
        idx = idx_v[pl.ds(j * _L, _L)]
        beta = plsc.load_gather(betas_v, [idx])
        alpha = plsc.load_gather(alphas_v, [idx])
        bar = plsc.load_gather(bars_v, [idx])
        bar = jnp.minimum(jnp.maximum(bar, 0.0), 1.0)
        p = lanes3 + j * (_L * 3)
        plsc.store_scatter(out_v, [p], beta)
        plsc.store_scatter(out_v, [p + 1], alpha)
        plsc.store_scatter(out_v, [p + 2], bar)
        return carry

    lax.fori_loop(0, _BPW // _L, step, 0, unroll=4)

    pltpu.sync_copy(out_v, out_hbm.at[pl.ds(base * 3, _BPW * 3)])


_ddpm_lookup = functools.partial(
    pl.kernel,
    out_type=jax.ShapeDtypeStruct((_B * 3,), jnp.float32),
    mesh=plsc.VectorSubcoreMesh(core_axis_name="c", subcore_axis_name="s", num_cores=1),
    compiler_params=pltpu.CompilerParams(needs_layout_passes=False),
    scratch_types=[
        pltpu.VMEM((_BPW,), jnp.int32),
        pltpu.VMEM((_TABLE,), jnp.float32),
        pltpu.VMEM((_TABLE,), jnp.float32),
        pltpu.VMEM((_TABLE,), jnp.float32),
        pltpu.VMEM((_BPW * 3,), jnp.float32),
        pltpu.SemaphoreType.DMA,
    ],
)(_body)


@jax.jit
def kernel(T, all_betas, all_alphas, all_bar_alphas):
    flat = _ddpm_lookup(T, all_betas, all_alphas, all_bar_alphas)
    return flat.reshape(_B, 3)
